# baseline (speedup 1.0000x reference)
import functools

import numpy as np

import jax
import jax.numpy as jnp
from jax import lax
from jax.experimental import pallas as pl
from jax.experimental.pallas import tpu as pltpu
from jax.experimental.pallas import tpu_sc as plsc

N = 131072
D = 64
K = 8 * 2 * 1024
NROW = 1024
NCOL = 128
PAD = N
PADN = K + PAD

NUM_CORES = 2
NUM_SUBCORES = 16
NW = NUM_CORES * NUM_SUBCORES
RPW = NROW // NW
EPW = N // NW

_NEG_INF_KEY = np.uint32(0x007FFFFF)


PACK = 8


def _select_compute(ks):
    t = jnp.uint32(0)
    for bit in range(31, -1, -1):
        cand = t | jnp.uint32(1 << bit)
        cand_s = lax.bitcast_convert_type(cand ^ jnp.uint32(0x80000000),
                                          jnp.int32)
        cnt = jnp.sum((ks >= cand_s).astype(jnp.int32))
        t = jnp.where(cnt >= K, cand, t)
    t_s = lax.bitcast_convert_type(t ^ jnp.uint32(0x80000000), jnp.int32)

    gt = ks > t_s
    eq = ks == t_s
    need = K - jnp.sum(gt.astype(jnp.int32))

    ri = lax.broadcasted_iota(jnp.int32, (NCOL, NCOL), 0)
    ci = lax.broadcasted_iota(jnp.int32, (NCOL, NCOL), 1)
    u_strict = (ri < ci).astype(jnp.float32)
    rl = lax.broadcasted_iota(jnp.int32, (NROW, NROW), 0)
    cl = lax.broadcasted_iota(jnp.int32, (NROW, NROW), 1)
    l_strict = (cl < rl).astype(jnp.float32)

    def exprefix(m):
        inrow = jnp.dot(m, u_strict, preferred_element_type=jnp.float32)
        rs = jnp.sum(m, axis=1, keepdims=True)
        rowpre = jnp.dot(l_strict, rs, preferred_element_type=jnp.float32)
        return inrow + rowpre

    eq_f = eq.astype(jnp.float32)
    eq_rank = exprefix(eq_f).astype(jnp.int32)
    sel = gt | (eq & (eq_rank < need))
    pos = exprefix(sel.astype(jnp.float32)).astype(jnp.int32)

    r_idx = lax.broadcasted_iota(jnp.int32, (NROW, NCOL), 0)
    c_idx = lax.broadcasted_iota(jnp.int32, (NROW, NCOL), 1)
    flat = r_idx * NCOL + c_idx
    return jnp.where(sel, pos, K + flat)


def _matvec8_body(x_ref, w_ref, o_ref):
    jj = lax.broadcasted_iota(jnp.int32, (1, PACK), 1)
    w_col = w_ref[...]
    acc = jnp.zeros(o_ref.shape, jnp.float32)
    for j in range(PACK):
        w_j = w_col * (jj == j).astype(jnp.float32)
        acc = acc + jnp.dot(x_ref[:, j, :], w_j,
                            preferred_element_type=jnp.float32)
    o_ref[...] = acc


def _scores(x, w):
    x3 = x.reshape(N // PACK, PACK, D)
    rows = N // PACK
    return pl.pallas_call(
        _matvec8_body,
        grid=(4,),
        in_specs=[
            pl.BlockSpec((rows // 4, PACK, D), lambda i: (i, 0, 0)),
            pl.BlockSpec((D, 1), lambda i: (0, 0)),
        ],
        out_specs=pl.BlockSpec((rows // 4, PACK), lambda i: (i, 0)),
        out_shape=jax.ShapeDtypeStruct((rows, PACK), jnp.float32),
    )(x3, w.reshape(D, 1))


def _select_body(s_ref, dst_ref):
    s = s_ref[...]
    b = lax.bitcast_convert_type(s, jnp.uint32)
    ku = jnp.where(b >> 31 == jnp.uint32(1), ~b, b | jnp.uint32(0x80000000))
    ku = jnp.where(jnp.isnan(s), _NEG_INF_KEY, ku)
    ks = lax.bitcast_convert_type(ku ^ jnp.uint32(0x80000000), jnp.int32)
    dst_ref[...] = _select_compute(ks)


def _select(scores2d):
    return pl.pallas_call(
        _select_body,
        out_shape=jax.ShapeDtypeStruct((NROW, NCOL), jnp.int32),
    )(scores2d)


def _sc_select_gather(x, dst2d):
    mesh = plsc.VectorSubcoreMesh(core_axis_name="c", subcore_axis_name="s")
    rps = NROW // NUM_SUBCORES
    opw = K // NW

    @functools.partial(
        pl.kernel,
        mesh=mesh,
        compiler_params=pltpu.CompilerParams(use_tc_tiling_on_sc=False),
        out_type=jax.ShapeDtypeStruct((K, D), jnp.float32),
        scratch_types=[
            pltpu.VMEM((rps, NCOL), jnp.int32),
            pltpu.VMEM((rps, NCOL), jnp.int32),
            pltpu.VMEM_SHARED((PADN,), jnp.int32),
            pltpu.VMEM((opw,), jnp.int32),
            pltpu.VMEM((opw, D), jnp.float32),
            pltpu.SemaphoreType.DMA,
        ],
    )
    def k(x_hbm, dst_hbm, out_hbm, dst_v, vals_v, idx_sh, idx_v, rows_v, sem):
        sid = lax.axis_index("s")
        wid = sid * NUM_CORES + lax.axis_index("c")
        row0 = sid * rps
        pltpu.sync_copy(dst_hbm.at[pl.ds(row0, rps)], dst_v)
        base = row0 * NCOL

        def fill(r, carry):
            first = base + r * NCOL
            for q in range(NCOL // 16):
                vals_v[r, pl.ds(q * 16, 16)] = (
                    first + q * 16 + lax.iota(jnp.int32, 16))
            return carry

        lax.fori_loop(0, rps, fill, 0)
        sc_handles = [
            pltpu.async_copy(vals_v.at[r], idx_sh.at[dst_v.at[r]], sem)
            for r in range(rps)
        ]
        for h in sc_handles:
            h.wait()
        plsc.subcore_barrier()
        j0 = wid * opw
        pltpu.sync_copy(idx_sh.at[pl.ds(j0, opw)], idx_v)
        handles = [
            pltpu.async_copy(x_hbm.at[idx_v.at[pl.ds(ci * NCOL, NCOL)]],
                             rows_v.at[pl.ds(ci * NCOL, NCOL)], sem)
            for ci in range(opw // NCOL)
        ]
        for h in handles:
            h.wait()
        pltpu.sync_copy(rows_v, out_hbm.at[pl.ds(j0, opw)])

    return k(x, dst2d)


def kernel(x, w, block_idx):
    scores = _scores(x, w)
    dst = _select(scores.reshape(NROW, NCOL))
    out = _sc_select_gather(x, dst)
    return out


# speedup vs baseline: 1.0242x; 1.0242x over previous
import functools

import numpy as np

import jax
import jax.numpy as jnp
from jax import lax
from jax.experimental import pallas as pl
from jax.experimental.pallas import tpu as pltpu
from jax.experimental.pallas import tpu_sc as plsc

N = 131072
D = 64
K = 8 * 2 * 1024
NROW = 1024
NCOL = 128
PAD = N
PADN = K + PAD

NUM_CORES = 2
NUM_SUBCORES = 16
NW = NUM_CORES * NUM_SUBCORES
RPW = NROW // NW
EPW = N // NW

_NEG_INF_KEY = np.uint32(0x007FFFFF)


PACK = 8


def _select_compute(ks):
    t = jnp.uint32(0)
    for bit in range(31, -1, -1):
        cand = t | jnp.uint32(1 << bit)
        cand_s = lax.bitcast_convert_type(cand ^ jnp.uint32(0x80000000),
                                          jnp.int32)
        cnt = jnp.sum((ks >= cand_s).astype(jnp.int32))
        t = jnp.where(cnt >= K, cand, t)
    t_s = lax.bitcast_convert_type(t ^ jnp.uint32(0x80000000), jnp.int32)

    gt = ks > t_s
    eq = ks == t_s
    need = K - jnp.sum(gt.astype(jnp.int32))

    ri = lax.broadcasted_iota(jnp.int32, (NCOL, NCOL), 0)
    ci = lax.broadcasted_iota(jnp.int32, (NCOL, NCOL), 1)
    u_strict = (ri < ci).astype(jnp.float32)
    rl = lax.broadcasted_iota(jnp.int32, (NROW, NROW), 0)
    cl = lax.broadcasted_iota(jnp.int32, (NROW, NROW), 1)
    l_strict = (cl < rl).astype(jnp.float32)

    def exprefix(m):
        inrow = jnp.dot(m, u_strict, preferred_element_type=jnp.float32)
        rs = jnp.sum(m, axis=1, keepdims=True)
        rowpre = jnp.dot(l_strict, rs, preferred_element_type=jnp.float32)
        return inrow + rowpre

    eq_f = eq.astype(jnp.float32)
    eq_rank = exprefix(eq_f).astype(jnp.int32)
    sel = gt | (eq & (eq_rank < need))
    pos = exprefix(sel.astype(jnp.float32)).astype(jnp.int32)

    r_idx = lax.broadcasted_iota(jnp.int32, (NROW, NCOL), 0)
    c_idx = lax.broadcasted_iota(jnp.int32, (NROW, NCOL), 1)
    flat = r_idx * NCOL + c_idx
    return jnp.where(sel, pos, K + flat)


def _matvec8_body(x_ref, w_ref, o_ref):
    jj = lax.broadcasted_iota(jnp.int32, (1, PACK), 1)
    w_col = w_ref[...]
    acc = jnp.zeros(o_ref.shape, jnp.float32)
    for j in range(PACK):
        w_j = w_col * (jj == j).astype(jnp.float32)
        acc = acc + jnp.dot(x_ref[:, j, :], w_j,
                            preferred_element_type=jnp.float32)
    o_ref[...] = acc


def _scores(x, w):
    x3 = x.reshape(N // PACK, PACK, D)
    rows = N // PACK
    return pl.pallas_call(
        _matvec8_body,
        grid=(8,),
        in_specs=[
            pl.BlockSpec((rows // 8, PACK, D), lambda i: (i, 0, 0)),
            pl.BlockSpec((D, 1), lambda i: (0, 0)),
        ],
        out_specs=pl.BlockSpec((rows // 8, PACK), lambda i: (i, 0)),
        out_shape=jax.ShapeDtypeStruct((rows, PACK), jnp.float32),
    )(x3, w.reshape(D, 1))


def _select_body(s_ref, dst_ref):
    s = s_ref[...]
    b = lax.bitcast_convert_type(s, jnp.uint32)
    ku = jnp.where(b >> 31 == jnp.uint32(1), ~b, b | jnp.uint32(0x80000000))
    ku = jnp.where(jnp.isnan(s), _NEG_INF_KEY, ku)
    ks = lax.bitcast_convert_type(ku ^ jnp.uint32(0x80000000), jnp.int32)
    dst_ref[...] = _select_compute(ks)


def _select(scores2d):
    return pl.pallas_call(
        _select_body,
        out_shape=jax.ShapeDtypeStruct((NROW, NCOL), jnp.int32),
    )(scores2d)


def _sc_select_gather(x, dst2d):
    mesh = plsc.VectorSubcoreMesh(core_axis_name="c", subcore_axis_name="s")
    rps = NROW // NUM_SUBCORES
    opw = K // NW

    @functools.partial(
        pl.kernel,
        mesh=mesh,
        compiler_params=pltpu.CompilerParams(use_tc_tiling_on_sc=False),
        out_type=jax.ShapeDtypeStruct((K, D), jnp.float32),
        scratch_types=[
            pltpu.VMEM((rps, NCOL), jnp.int32),
            pltpu.VMEM((rps, NCOL), jnp.int32),
            pltpu.VMEM_SHARED((PADN,), jnp.int32),
            pltpu.VMEM((opw,), jnp.int32),
            pltpu.VMEM((opw, D), jnp.float32),
            pltpu.SemaphoreType.DMA,
        ],
    )
    def k(x_hbm, dst_hbm, out_hbm, dst_v, vals_v, idx_sh, idx_v, rows_v, sem):
        sid = lax.axis_index("s")
        wid = sid * NUM_CORES + lax.axis_index("c")
        row0 = sid * rps
        pltpu.sync_copy(dst_hbm.at[pl.ds(row0, rps)], dst_v)
        base = row0 * NCOL

        def fill(r, carry):
            first = base + r * NCOL
            for q in range(NCOL // 16):
                vals_v[r, pl.ds(q * 16, 16)] = (
                    first + q * 16 + lax.iota(jnp.int32, 16))
            return carry

        lax.fori_loop(0, rps, fill, 0)
        sc_handles = [
            pltpu.async_copy(vals_v.at[r], idx_sh.at[dst_v.at[r]], sem)
            for r in range(rps)
        ]
        for h in sc_handles:
            h.wait()
        plsc.subcore_barrier()
        j0 = wid * opw
        pltpu.sync_copy(idx_sh.at[pl.ds(j0, opw)], idx_v)
        handles = [
            pltpu.async_copy(x_hbm.at[idx_v.at[pl.ds(ci * NCOL, NCOL)]],
                             rows_v.at[pl.ds(ci * NCOL, NCOL)], sem)
            for ci in range(opw // NCOL)
        ]
        for h in handles:
            h.wait()
        pltpu.sync_copy(rows_v, out_hbm.at[pl.ds(j0, opw)])

    return k(x, dst2d)


def kernel(x, w, block_idx):
    scores = _scores(x, w)
    dst = _select(scores.reshape(NROW, NCOL))
    out = _sc_select_gather(x, dst)
    return out
